# Initial kernel scaffold; baseline (speedup 1.0000x reference)
import jax
import jax.numpy as jnp
from jax.experimental import pallas as pl


def kernel(model, edge_index):
    raise NotImplementedError("write your pallas kernel here")


# speedup vs baseline: 1.1015x; 1.1015x over previous
import functools

import jax
import jax.numpy as jnp
from jax import lax
from jax.experimental import pallas as pl
from jax.experimental.pallas import tpu as pltpu
from jax.experimental.pallas import tpu_sc as plsc

E = 320000
D = 128
NC = 2
NS = 16
NW = NC * NS
B_W = E // NW
C = 80
NCHUNK = B_W // C

_mesh = plsc.VectorSubcoreMesh(core_axis_name="c", subcore_axis_name="s")


@functools.partial(
    pl.kernel,
    mesh=_mesh,
    compiler_params=pltpu.CompilerParams(needs_layout_passes=False),
    out_type=jax.ShapeDtypeStruct((E,), jnp.float32),
    scratch_types=[
        pltpu.VMEM((C,), jnp.int32),
        pltpu.VMEM((C,), jnp.int32),
        pltpu.VMEM((C, D), jnp.float32),
        pltpu.VMEM((C, D), jnp.float32),
        pltpu.VMEM((C,), jnp.float32),
        pltpu.SemaphoreType.DMA,
        pltpu.SemaphoreType.DMA,
    ],
)
def _edge_dot(table_hbm, i0_hbm, i1_hbm, out_hbm,
              idx0_v, idx1_v, ra_v, rb_v, o_v, sem0, sem1):
    wid = lax.axis_index("s") * NC + lax.axis_index("c")
    base = wid * B_W
    lanes = lax.iota(jnp.int32, 16)

    def chunk_body(g, carry):
        off = base + g * C
        pltpu.sync_copy(i0_hbm.at[pl.ds(off, C)], idx0_v)
        pltpu.sync_copy(i1_hbm.at[pl.ds(off, C)], idx1_v)
        cpa = pltpu.async_copy(table_hbm.at[idx0_v], ra_v, sem0)
        cpb = pltpu.async_copy(table_hbm.at[idx1_v], rb_v, sem1)
        cpa.wait()
        cpb.wait()

        def grp_body(grp, carry2):
            e0 = grp * 16
            ridx = e0 + lanes

            def d_body(dk, acc):
                for u in range(16):
                    col = jnp.full((16,), dk * 16 + u, jnp.int32)
                    a = plsc.load_gather(ra_v, [ridx, col])
                    b = plsc.load_gather(rb_v, [ridx, col])
                    acc = acc + a * b
                return acc

            acc = lax.fori_loop(0, D // 16, d_body,
                                jnp.zeros((16,), jnp.float32))
            o_v[pl.ds(e0, 16)] = acc
            return carry2

        lax.fori_loop(0, C // 16, grp_body, 0)
        pltpu.sync_copy(o_v, out_hbm.at[pl.ds(off, C)])
        return carry

    lax.fori_loop(0, NCHUNK, chunk_body, 0)


def kernel(model, edge_index):
    ei = edge_index.astype(jnp.int32)
    return _edge_dot(model, ei[0], ei[1])


# speedup vs baseline: 4.5106x; 4.0951x over previous
import functools

import jax
import jax.numpy as jnp
from jax import lax
from jax.experimental import pallas as pl
from jax.experimental.pallas import tpu as pltpu
from jax.experimental.pallas import tpu_sc as plsc

E = 320000
D = 128
NC = 2
NS = 16
NW = NC * NS
C = 128
NCH = E // C
TMAX = -(-NCH // NW)
NBUF = 2

_mesh = plsc.VectorSubcoreMesh(core_axis_name="c", subcore_axis_name="s")


@functools.partial(
    pl.kernel,
    mesh=_mesh,
    compiler_params=pltpu.CompilerParams(needs_layout_passes=False),
    out_type=jax.ShapeDtypeStruct((E,), jnp.float32),
    scratch_types=[
        pltpu.VMEM((16,), jnp.int32),
        pltpu.VMEM((TMAX + 1, C), jnp.int32),
        pltpu.VMEM((TMAX + 1, C), jnp.int32),
        [pltpu.VMEM((C, D), jnp.float32) for _ in range(NBUF)],
        [pltpu.VMEM((C, D), jnp.float32) for _ in range(NBUF)],
        [pltpu.VMEM((C,), jnp.float32) for _ in range(NBUF)],
        [pltpu.SemaphoreType.DMA for _ in range(NBUF)],
        [pltpu.SemaphoreType.DMA for _ in range(NBUF)],
        [pltpu.SemaphoreType.DMA for _ in range(NBUF)],
        pltpu.SemaphoreType.DMA,
    ],
)
def _edge_dot(i0_hbm, i1_hbm, table_hbm, out_hbm,
              csel_v, idx0_v, idx1_v, ra, rb, ov, sa, sb, so, si):
    wid = lax.axis_index("s") * NC + lax.axis_index("c")
    lanes = lax.iota(jnp.int32, 16)
    n_w = jnp.where(wid < NCH - TMAX * NW + NW, TMAX, TMAX - 1)

    def stage_idx(q, carry):
        csel_v[...] = jnp.minimum(wid + NW * (16 * q + lanes), NCH - 1)
        cp0 = pltpu.async_copy(
            i0_hbm.at[csel_v], idx0_v.at[pl.ds(16 * q, 16)], si)
        cp0.wait()
        cp1 = pltpu.async_copy(
            i1_hbm.at[csel_v], idx1_v.at[pl.ds(16 * q, 16)], si)
        cp1.wait()
        return carry

    lax.fori_loop(0, (TMAX + 1) // 16, stage_idx, 0)

    def issue(t, b):
        pltpu.async_copy(table_hbm.at[idx0_v.at[t]], ra[b], sa[b])
        pltpu.async_copy(table_hbm.at[idx1_v.at[t]], rb[b], sb[b])

    def compute(t, b):
        pltpu.make_async_copy(table_hbm.at[idx0_v.at[t]], ra[b], sa[b]).wait()
        pltpu.make_async_copy(table_hbm.at[idx1_v.at[t]], rb[b], sb[b]).wait()

        @pl.when(t >= NBUF)
        def _():
            pltpu.make_async_copy(
                ov[b], out_hbm.at[pl.ds(0, C)], so[b]).wait()

        def grp(gi, carry):
            e0 = gi * 16
            outv = jnp.zeros((16,), jnp.float32)
            for j in range(16):
                e = e0 + j
                p = [ra[b][e, pl.ds(16 * k, 16)] * rb[b][e, pl.ds(16 * k, 16)]
                     for k in range(8)]
                s01, s23 = p[0] + p[1], p[2] + p[3]
                s45, s67 = p[4] + p[5], p[6] + p[7]
                s = jnp.sum((s01 + s23) + (s45 + s67))
                outv = jnp.where(lanes == j, s, outv)
            ov[b][pl.ds(e0, 16)] = outv
            return carry

        lax.fori_loop(0, C // 16, grp, 0)
        off = pl.multiple_of((wid + NW * t) * C, C)
        pltpu.async_copy(ov[b], out_hbm.at[pl.ds(off, C)], so[b])

    issue(0, 0)

    def ring(i, carry):
        g = i * NBUF
        for b in range(NBUF):
            t = g + b

            @pl.when(t + 1 < n_w)
            def _():
                issue(t + 1, (b + 1) % NBUF)

            @pl.when(t < n_w)
            def _():
                compute(t, b)
        return carry

    lax.fori_loop(0, -(-TMAX // NBUF), ring, 0)

    for b in range(NBUF):
        pltpu.make_async_copy(ov[b], out_hbm.at[pl.ds(0, C)], so[b]).wait()


def kernel(model, edge_index):
    ei = edge_index.astype(jnp.int32)
    i0 = ei[0].reshape(NCH, C)
    i1 = ei[1].reshape(NCH, C)
    return _edge_dot(i0, i1, model)
